# baseline (speedup 1.0000x reference)
import functools

import jax
import jax.numpy as jnp
from jax import lax
from jax.experimental import pallas as pl
from jax.experimental.pallas import tpu as pltpu
from jax.experimental.pallas import tpu_sc as plsc

_VOCAB = 1000000
_EMBED = 64
_OUT = 3
_DPAD = 16
_BATCH = 16384
_HIST = 50
_NIDX = _BATCH * _HIST

_NC, _NS = 2, 16
_NW = _NC * _NS
_BPW = _NIDX // _NW
_CHUNK = 3200
_NCHUNK = _BPW // _CHUNK

_MM_ROWS = 8000


def _mm_body(t_ref, w_ref, b_ref, o_ref):
    o_ref[...] = (
        jnp.dot(t_ref[...], w_ref[...], preferred_element_type=jnp.float32)
        + b_ref[...]
    )


def _project_table(emb_table, w_pad, b_pad):
    grid = _VOCAB // _MM_ROWS
    return pl.pallas_call(
        _mm_body,
        grid=(grid,),
        in_specs=[
            pl.BlockSpec((_MM_ROWS, _EMBED), lambda i: (i, 0)),
            pl.BlockSpec((_EMBED, _DPAD), lambda i: (0, 0)),
            pl.BlockSpec((1, _DPAD), lambda i: (0, 0)),
        ],
        out_specs=pl.BlockSpec((_MM_ROWS, _DPAD), lambda i: (i, 0)),
        out_shape=jax.ShapeDtypeStruct((_VOCAB, _DPAD), jnp.float32),
    )(emb_table, w_pad, b_pad)


_sc_mesh = plsc.VectorSubcoreMesh(core_axis_name="c", subcore_axis_name="s")

_GRP = _CHUNK // 16
_UNROLL = 4


@functools.partial(
    pl.kernel,
    mesh=_sc_mesh,
    compiler_params=pltpu.CompilerParams(
        use_tc_tiling_on_sc=False, needs_layout_passes=False
    ),
    out_type=jax.ShapeDtypeStruct((_NIDX, _OUT), jnp.float32),
    scratch_types=[
        pltpu.VMEM((_CHUNK,), jnp.int32),
        pltpu.VMEM((_CHUNK, _DPAD), jnp.float32),
        pltpu.VMEM((_CHUNK, _OUT), jnp.float32),
        pltpu.SemaphoreType.DMA,
    ],
)
def _gather_sc(proj_hbm, idx_hbm, out_hbm, idx_v, rows_v, out3_v, sem):
    wid = lax.axis_index("s") * _NC + lax.axis_index("c")
    base = wid * _BPW
    lanes = lax.iota(jnp.int32, 16)
    for ci in range(_NCHUNK):
        off = base + ci * _CHUNK
        pltpu.sync_copy(idx_hbm.at[pl.ds(off, _CHUNK)], idx_v)
        pltpu.async_copy(proj_hbm.at[idx_v], rows_v, sem).wait()

        def _grp(t, _):
            for u in range(_UNROLL):
                g_vec = (t * _UNROLL + u) * 16 + lanes
                for c in range(_OUT):
                    c_vec = jnp.full((16,), c, jnp.int32)
                    vals = plsc.load_gather(rows_v, [g_vec, c_vec])
                    plsc.store_scatter(out3_v, [g_vec, c_vec], vals)
            return _

        lax.fori_loop(0, _GRP // _UNROLL, _grp, 0)
        pltpu.sync_copy(out3_v, out_hbm.at[pl.ds(off, _CHUNK)])


def kernel(x, emb_table, W, b):
    w_pad = jnp.zeros((_EMBED, _DPAD), jnp.float32).at[:, :_OUT].set(W)
    b_pad = jnp.zeros((1, _DPAD), jnp.float32).at[0, :_OUT].set(b)
    proj = _project_table(emb_table, w_pad, b_pad)
    out3 = _gather_sc(proj, x.reshape(-1))
    return out3.reshape(_BATCH, _HIST, _OUT)


# speedup vs baseline: 5.6036x; 5.6036x over previous
import functools

import jax
import jax.numpy as jnp
from jax import lax
from jax.experimental import pallas as pl
from jax.experimental.pallas import tpu as pltpu
from jax.experimental.pallas import tpu_sc as plsc

_VOCAB = 1000000
_EMBED = 64
_OUT = 3
_DPAD = 8
_BATCH = 16384
_HIST = 50
_NIDX = _BATCH * _HIST

_NC, _NS = 2, 16
_NW = _NC * _NS
_BPW = _NIDX // _NW
_CHUNK = 3200
_NCHUNK = _BPW // _CHUNK

_MM_COLS = 32768


def _mm_body(t_ref, w_ref, b_ref, o_ref):
    o_ref[...] = lax.dot_general(
        w_ref[...], t_ref[...], (((0,), (0,)), ((), ())),
        preferred_element_type=jnp.float32,
    ) + b_ref[...]


def _project_table(emb_t, w_pad, b_pad):
    grid = (_VOCAB + _MM_COLS - 1) // _MM_COLS
    return pl.pallas_call(
        _mm_body,
        grid=(grid,),
        in_specs=[
            pl.BlockSpec((_EMBED, _MM_COLS), lambda i: (0, i)),
            pl.BlockSpec((_EMBED, _DPAD), lambda i: (0, 0)),
            pl.BlockSpec((_DPAD, 1), lambda i: (0, 0)),
        ],
        out_specs=pl.BlockSpec((_DPAD, _MM_COLS), lambda i: (0, i)),
        out_shape=jax.ShapeDtypeStruct((_DPAD, _VOCAB), jnp.float32),
    )(emb_t, w_pad, b_pad)


_sc_mesh = plsc.VectorSubcoreMesh(core_axis_name="c", subcore_axis_name="s")


@functools.partial(
    pl.kernel,
    mesh=_sc_mesh,
    compiler_params=pltpu.CompilerParams(
        use_tc_tiling_on_sc=False, needs_layout_passes=False
    ),
    out_type=jax.ShapeDtypeStruct((_OUT, _NIDX), jnp.float32),
    scratch_types=[
        pltpu.VMEM((_CHUNK,), jnp.int32),
        pltpu.VMEM((_OUT, _CHUNK), jnp.float32),
        pltpu.SemaphoreType.DMA,
    ],
)
def _gather_sc(p0, p1, p2, idx_hbm, out_hbm, idx_v, vals_v, sem):
    wid = lax.axis_index("s") * _NC + lax.axis_index("c")
    base = wid * _BPW
    planes = (p0, p1, p2)
    for ci in range(_NCHUNK):
        off = base + ci * _CHUNK
        pltpu.sync_copy(idx_hbm.at[pl.ds(off, _CHUNK)], idx_v)
        copies = [
            pltpu.async_copy(planes[c].at[idx_v], vals_v.at[c], sem)
            for c in range(_OUT)
        ]
        for c in copies:
            c.wait()
        for c in range(_OUT):
            pltpu.sync_copy(vals_v.at[c], out_hbm.at[c, pl.ds(off, _CHUNK)])


def kernel(x, emb_table, W, b):
    w_pad = jnp.zeros((_EMBED, _DPAD), jnp.float32).at[:, :_OUT].set(W)
    b_pad = jnp.zeros((_DPAD, 1), jnp.float32).at[:_OUT, 0].set(b)
    proj_t = _project_table(emb_table.T, w_pad, b_pad)
    idx_lin = x.T.reshape(-1)
    out2 = _gather_sc(proj_t[0], proj_t[1], proj_t[2], idx_lin)
    return out2.reshape(_OUT, _HIST, _BATCH).transpose(2, 1, 0)
